# Initial kernel scaffold; baseline (speedup 1.0000x reference)
import jax
import jax.numpy as jnp
from jax.experimental import pallas as pl


def kernel(x, batch_ids, gate_W1, gate_b1, gate_ln_g, gate_ln_b, gate_W2, gate_b2, pool_ln_g, pool_ln_b, mlp_W1, mlp_b1, mlp_ln_g, mlp_ln_b, mlp_W2, mlp_b2):
    raise NotImplementedError("write your pallas kernel here")


# speedup vs baseline: 9.6757x; 9.6757x over previous
import functools

import jax
import jax.numpy as jnp
from jax.experimental import pallas as pl
from jax.experimental.pallas import tpu as pltpu

N = 100000
D = 128
H = 256
OUT = 250
S = 256

B = 1024
NB = (N + B - 1) // B
NEG = -1e30


def _ln(v, g, b, axis, eps=1e-5):
    mu = jnp.mean(v, axis=axis, keepdims=True)
    var = jnp.mean((v - mu) * (v - mu), axis=axis, keepdims=True)
    return (v - mu) * jax.lax.rsqrt(var + eps) * g + b


def _body(x_ref, ids_ref, gw1_ref, gb1_ref, glg_ref, glb_ref, gw2_ref, gb2_ref,
          plg_ref, plb_ref, mw1_ref, mb1_ref, mlg_ref, mlb_ref, mw2_ref, mb2_ref,
          out_ref, m_ref, d_ref, u_ref):
    i = pl.program_id(0)

    @pl.when(i == 0)
    def _init():
        m_ref[...] = jnp.full((S, 1), NEG, jnp.float32)
        d_ref[...] = jnp.zeros((S, 1), jnp.float32)
        u_ref[...] = jnp.zeros((S, D), jnp.float32)

    xb = x_ref[...]
    rowi = jax.lax.broadcasted_iota(jnp.int32, (B, 1), 0)
    xb = jnp.where(i * B + rowi < N, xb, 0.0)

    h = jnp.dot(xb, gw1_ref[...], preferred_element_type=jnp.float32) + gb1_ref[...]
    h = _ln(h, glg_ref[...], glb_ref[...], axis=1)
    h = jnp.maximum(h, 0.0)
    gate_col = jnp.sum(h * gw2_ref[...], axis=1, keepdims=True) + gb2_ref[0, 0]
    gate_row = gate_col.T

    ids_row = ids_ref[0]
    lane = jax.lax.broadcasted_iota(jnp.int32, (1, B), 1)
    valid = (i * B + lane) < N
    iota_s = jax.lax.broadcasted_iota(jnp.int32, (S, B), 0)
    eq = (iota_s == ids_row) & valid
    gate_row = jnp.where(valid, gate_row, NEG)

    bmax = jnp.max(jnp.where(eq, gate_row, NEG), axis=1, keepdims=True)
    m_old = m_ref[...]
    m_new = jnp.maximum(m_old, bmax)
    scale = jnp.exp(m_old - m_new)
    m_ref[...] = m_new

    m_sel = jnp.sum(jnp.where(eq, m_new, 0.0), axis=0, keepdims=True)
    e_row = jnp.exp(gate_row - m_sel)
    p0e = eq.astype(jnp.float32) * e_row
    d_ref[...] = d_ref[...] * scale + jnp.sum(p0e, axis=1, keepdims=True)
    u_ref[...] = u_ref[...] * scale + jnp.dot(
        p0e, xb, preferred_element_type=jnp.float32)

    @pl.when(i == NB - 1)
    def _fin():
        pooled = u_ref[...] / (d_ref[...] + 1e-16)
        pooled = _ln(pooled, plg_ref[...], plb_ref[...], axis=1)
        z = jnp.dot(pooled, mw1_ref[...], preferred_element_type=jnp.float32) + mb1_ref[...]
        z = jnp.maximum(_ln(z, mlg_ref[...], mlb_ref[...], axis=1), 0.0)
        out_ref[...] = jnp.dot(z, mw2_ref[...], preferred_element_type=jnp.float32) + mb2_ref[...]


@functools.partial(jax.jit, static_argnames=("interpret",))
def _run(x, ids_r, gw1, gb1, glg, glb, gw2r, gb2, plg, plb, mw1, mb1, mlg, mlb,
         mw2p, mb2p, interpret=False):
    full = lambda *shape: pl.BlockSpec(shape, lambda i: (0,) * len(shape))
    return pl.pallas_call(
        _body,
        grid=(NB,),
        in_specs=[
            pl.BlockSpec((B, D), lambda i: (i, 0)),
            pl.BlockSpec((1, 1, B), lambda i: (i, 0, 0)),
            full(D, 2 * D), full(1, 2 * D), full(1, 2 * D), full(1, 2 * D),
            full(1, 2 * D), full(1, 1),
            full(1, D), full(1, D),
            full(D, H), full(1, H), full(1, H), full(1, H),
            full(H, 256), full(1, 256),
        ],
        out_specs=pl.BlockSpec((S, 256), lambda i: (0, 0)),
        out_shape=jax.ShapeDtypeStruct((S, 256), jnp.float32),
        scratch_shapes=[
            pltpu.VMEM((S, 1), jnp.float32),
            pltpu.VMEM((S, 1), jnp.float32),
            pltpu.VMEM((S, D), jnp.float32),
        ],
        compiler_params=pltpu.CompilerParams(
            dimension_semantics=("arbitrary",)),
        interpret=interpret,
    )(x, ids_r, gw1, gb1, glg, glb, gw2r, gb2, plg, plb, mw1, mb1, mlg, mlb,
      mw2p, mb2p)


def kernel(x, batch_ids, gate_W1, gate_b1, gate_ln_g, gate_ln_b, gate_W2,
           gate_b2, pool_ln_g, pool_ln_b, mlp_W1, mlp_b1, mlp_ln_g, mlp_ln_b,
           mlp_W2, mlp_b2, interpret=False):
    ids_p = jnp.pad(batch_ids, (0, NB * B - N), constant_values=S)
    ids_r = ids_p.reshape(NB, 1, B)
    row = lambda v: v.reshape(1, -1)
    mw2p = jnp.pad(mlp_W2, ((0, 0), (0, 256 - OUT)))
    mb2p = jnp.pad(mlp_b2, (0, 256 - OUT)).reshape(1, 256)
    logits = _run(x, ids_r, gate_W1, row(gate_b1), row(gate_ln_g),
                  row(gate_ln_b), gate_W2.reshape(1, 2 * D),
                  gate_b2.reshape(1, 1), row(pool_ln_g), row(pool_ln_b),
                  mlp_W1, row(mlp_b1), row(mlp_ln_g), row(mlp_ln_b),
                  mw2p, mb2p, interpret=interpret)
    return logits[:, :OUT]


# speedup vs baseline: 11.8223x; 1.2219x over previous
import functools

import jax
import jax.numpy as jnp
from jax.experimental import pallas as pl
from jax.experimental.pallas import tpu as pltpu

N = 100000
D = 128
H = 256
OUT = 250
S = 256

B = 1024
NB = (N + B - 1) // B
NEG = -1e30


def _ln(v, g, b, axis, eps=1e-5):
    mu = jnp.mean(v, axis=axis, keepdims=True)
    var = jnp.mean((v - mu) * (v - mu), axis=axis, keepdims=True)
    return (v - mu) * jax.lax.rsqrt(var + eps) * g + b


def _body(x_ref, ids_ref, gw1t_ref, gb1_ref, glg_ref, glb_ref, gw2_ref, gb2_ref,
          plg_ref, plb_ref, mw1_ref, mb1_ref, mlg_ref, mlb_ref, mw2_ref, mb2_ref,
          out_ref, m_ref, d_ref, u_ref):
    i = pl.program_id(0)

    @pl.when(i == 0)
    def _init():
        m_ref[...] = jnp.full((S, 1), NEG, jnp.float32)
        d_ref[...] = jnp.zeros((S, 1), jnp.float32)
        u_ref[...] = jnp.zeros((S, D), jnp.float32)

    xb = x_ref[...]
    rowi = jax.lax.broadcasted_iota(jnp.int32, (B, 1), 0)
    xb = jnp.where(i * B + rowi < N, xb, 0.0)

    ht = jnp.dot(gw1t_ref[...], xb.T,
                 preferred_element_type=jnp.float32) + gb1_ref[...]
    ht = _ln(ht, glg_ref[...], glb_ref[...], axis=0)
    ht = jnp.maximum(ht, 0.0)
    gate_row = jnp.sum(ht * gw2_ref[...], axis=0, keepdims=True) + gb2_ref[0, 0]

    ids_row = ids_ref[0]
    iota_s = jax.lax.broadcasted_iota(jnp.int32, (S, B), 0)
    eq = iota_s == ids_row

    bmax = jnp.max(jnp.where(eq, gate_row, NEG), axis=1, keepdims=True)
    m_old = m_ref[...]
    m_new = jnp.maximum(m_old, bmax)
    scale = jnp.exp(m_old - m_new)
    m_ref[...] = m_new

    m_sel = jnp.sum(jnp.where(eq, m_new, 0.0), axis=0, keepdims=True)
    e_row = jnp.exp(gate_row - m_sel)
    p0e = eq.astype(jnp.float32) * e_row
    d_ref[...] = d_ref[...] * scale + jnp.sum(p0e, axis=1, keepdims=True)
    u_ref[...] = u_ref[...] * scale + jnp.dot(
        p0e, xb, preferred_element_type=jnp.float32)

    @pl.when(i == NB - 1)
    def _fin():
        pooled = u_ref[...] / (d_ref[...] + 1e-16)
        pooled = _ln(pooled, plg_ref[...], plb_ref[...], axis=1)
        z = jnp.dot(pooled, mw1_ref[...], preferred_element_type=jnp.float32) + mb1_ref[...]
        z = jnp.maximum(_ln(z, mlg_ref[...], mlb_ref[...], axis=1), 0.0)
        out_ref[...] = jnp.dot(z, mw2_ref[...], preferred_element_type=jnp.float32) + mb2_ref[...]


@functools.partial(jax.jit, static_argnames=("interpret",))
def _run(x, ids_r, gw1t, gb1c, glgc, glbc, gw2c, gb2, plg, plb, mw1, mb1, mlg,
         mlb, mw2p, mb2p, interpret=False):
    full = lambda *shape: pl.BlockSpec(shape, lambda i: (0,) * len(shape))
    return pl.pallas_call(
        _body,
        grid=(NB,),
        in_specs=[
            pl.BlockSpec((B, D), lambda i: (i, 0)),
            pl.BlockSpec((1, 1, B), lambda i: (i, 0, 0)),
            full(2 * D, D), full(2 * D, 1), full(2 * D, 1), full(2 * D, 1),
            full(2 * D, 1), full(1, 1),
            full(1, D), full(1, D),
            full(D, H), full(1, H), full(1, H), full(1, H),
            full(H, 256), full(1, 256),
        ],
        out_specs=pl.BlockSpec((S, 256), lambda i: (0, 0)),
        out_shape=jax.ShapeDtypeStruct((S, 256), jnp.float32),
        scratch_shapes=[
            pltpu.VMEM((S, 1), jnp.float32),
            pltpu.VMEM((S, 1), jnp.float32),
            pltpu.VMEM((S, D), jnp.float32),
        ],
        compiler_params=pltpu.CompilerParams(
            dimension_semantics=("arbitrary",)),
        interpret=interpret,
    )(x, ids_r, gw1t, gb1c, glgc, glbc, gw2c, gb2, plg, plb, mw1, mb1, mlg,
      mlb, mw2p, mb2p)


def kernel(x, batch_ids, gate_W1, gate_b1, gate_ln_g, gate_ln_b, gate_W2,
           gate_b2, pool_ln_g, pool_ln_b, mlp_W1, mlp_b1, mlp_ln_g, mlp_ln_b,
           mlp_W2, mlp_b2, interpret=False):
    ids_p = jnp.pad(batch_ids, (0, NB * B - N), constant_values=S)
    ids_r = ids_p.reshape(NB, 1, B)
    row = lambda v: v.reshape(1, -1)
    col = lambda v: v.reshape(-1, 1)
    mw2p = jnp.pad(mlp_W2, ((0, 0), (0, 256 - OUT)))
    mb2p = jnp.pad(mlp_b2, (0, 256 - OUT)).reshape(1, 256)
    logits = _run(x, ids_r, gate_W1.T, col(gate_b1), col(gate_ln_g),
                  col(gate_ln_b), gate_W2.reshape(2 * D, 1),
                  gate_b2.reshape(1, 1), row(pool_ln_g), row(pool_ln_b),
                  mlp_W1, row(mlp_b1), row(mlp_ln_g), row(mlp_ln_b),
                  mw2p, mb2p, interpret=interpret)
    return logits[:, :OUT]


# speedup vs baseline: 13.5805x; 1.1487x over previous
import functools

import jax
import jax.numpy as jnp
from jax.experimental import pallas as pl
from jax.experimental.pallas import tpu as pltpu

N = 100000
D = 128
H = 256
OUT = 250
S = 256

B = 1024
NB = (N + B - 1) // B
W = 64
NEG = -1e30
PAD_ID = 511


def _ln(v, g, b, axis, eps=1e-5):
    mu = jnp.mean(v, axis=axis, keepdims=True)
    var = jnp.mean((v - mu) * (v - mu), axis=axis, keepdims=True)
    return (v - mu) * jax.lax.rsqrt(var + eps) * g + b


def _body(x_ref, ids_ref, idsm_ref, gw1t_ref, gb1_ref, glg_ref, glb_ref,
          gw2_ref, gb2_ref, plg_ref, plb_ref, mw1_ref, mb1_ref, mlg_ref,
          mlb_ref, mw2_ref, mb2_ref, out_ref, m_ref, d_ref, u_ref):
    i = pl.program_id(0)

    @pl.when(i == 0)
    def _init():
        m_ref[...] = jnp.full((S, 1), NEG, jnp.float32)
        d_ref[...] = jnp.zeros((S, 1), jnp.float32)
        u_ref[...] = jnp.zeros((S, D), jnp.float32)

    xb = x_ref[...]
    rowi = jax.lax.broadcasted_iota(jnp.int32, (B, 1), 0)
    xb = jnp.where(i * B + rowi < N, xb, 0.0)

    ht = jnp.dot(gw1t_ref[...], xb.T,
                 preferred_element_type=jnp.float32) + gb1_ref[...]
    ht = _ln(ht, glg_ref[...], glb_ref[...], axis=0)
    ht = jnp.maximum(ht, 0.0)
    gate_row = jnp.sum(ht * gw2_ref[...], axis=0, keepdims=True) + gb2_ref[0, 0]

    ids_row = ids_ref[0]
    iota_w = jax.lax.broadcasted_iota(jnp.int32, (W, B), 0)

    first = idsm_ref[0, 0, 0]
    last = jnp.minimum(idsm_ref[0, 0, B - 1], S - 1)
    w0 = first // W
    nwin = last // W - w0 + 1

    def _window(w, carry):
        base = (w0 + w) * W
        rel = ids_row - base
        eq = iota_w == rel
        bmax = jnp.max(jnp.where(eq, gate_row, NEG), axis=1, keepdims=True)
        sl = pl.ds(base, W)
        m_old = m_ref[sl, :]
        m_new = jnp.maximum(m_old, bmax)
        scale = jnp.exp(m_old - m_new)
        m_ref[sl, :] = m_new
        m_sel = jnp.sum(jnp.where(eq, m_new, 0.0), axis=0, keepdims=True)
        e_row = jnp.exp(jnp.minimum(gate_row - m_sel, 0.0))
        p0e = eq.astype(jnp.float32) * e_row
        d_ref[sl, :] = d_ref[sl, :] * scale + jnp.sum(p0e, axis=1, keepdims=True)
        u_ref[sl, :] = u_ref[sl, :] * scale + jnp.dot(
            p0e, xb, preferred_element_type=jnp.float32)
        return carry

    jax.lax.fori_loop(0, nwin, _window, 0, unroll=False)

    @pl.when(i == NB - 1)
    def _fin():
        pooled = u_ref[...] / (d_ref[...] + 1e-16)
        pooled = _ln(pooled, plg_ref[...], plb_ref[...], axis=1)
        z = jnp.dot(pooled, mw1_ref[...], preferred_element_type=jnp.float32) + mb1_ref[...]
        z = jnp.maximum(_ln(z, mlg_ref[...], mlb_ref[...], axis=1), 0.0)
        out_ref[...] = jnp.dot(z, mw2_ref[...], preferred_element_type=jnp.float32) + mb2_ref[...]


@functools.partial(jax.jit, static_argnames=("interpret",))
def _run(x, ids_r, gw1t, gb1c, glgc, glbc, gw2c, gb2, plg, plb, mw1, mb1, mlg,
         mlb, mw2p, mb2p, interpret=False):
    full = lambda *shape: pl.BlockSpec(shape, lambda i: (0,) * len(shape))
    return pl.pallas_call(
        _body,
        grid=(NB,),
        in_specs=[
            pl.BlockSpec((B, D), lambda i: (i, 0)),
            pl.BlockSpec((1, 1, B), lambda i: (i, 0, 0)),
            pl.BlockSpec((1, 1, B), lambda i: (i, 0, 0),
                         memory_space=pltpu.SMEM),
            full(2 * D, D), full(2 * D, 1), full(2 * D, 1), full(2 * D, 1),
            full(2 * D, 1), full(1, 1),
            full(1, D), full(1, D),
            full(D, H), full(1, H), full(1, H), full(1, H),
            full(H, 256), full(1, 256),
        ],
        out_specs=pl.BlockSpec((S, 256), lambda i: (0, 0)),
        out_shape=jax.ShapeDtypeStruct((S, 256), jnp.float32),
        scratch_shapes=[
            pltpu.VMEM((S, 1), jnp.float32),
            pltpu.VMEM((S, 1), jnp.float32),
            pltpu.VMEM((S, D), jnp.float32),
        ],
        compiler_params=pltpu.CompilerParams(
            dimension_semantics=("arbitrary",)),
        interpret=interpret,
    )(x, ids_r, ids_r, gw1t, gb1c, glgc, glbc, gw2c, gb2, plg, plb, mw1, mb1,
      mlg, mlb, mw2p, mb2p)


def kernel(x, batch_ids, gate_W1, gate_b1, gate_ln_g, gate_ln_b, gate_W2,
           gate_b2, pool_ln_g, pool_ln_b, mlp_W1, mlp_b1, mlp_ln_g, mlp_ln_b,
           mlp_W2, mlp_b2, interpret=False):
    ids_p = jnp.pad(batch_ids, (0, NB * B - N), constant_values=PAD_ID)
    ids_r = ids_p.reshape(NB, 1, B)
    row = lambda v: v.reshape(1, -1)
    col = lambda v: v.reshape(-1, 1)
    mw2p = jnp.pad(mlp_W2, ((0, 0), (0, 256 - OUT)))
    mb2p = jnp.pad(mlp_b2, (0, 256 - OUT)).reshape(1, 256)
    logits = _run(x, ids_r, gate_W1.T, col(gate_b1), col(gate_ln_g),
                  col(gate_ln_b), gate_W2.reshape(2 * D, 1),
                  gate_b2.reshape(1, 1), row(pool_ln_g), row(pool_ln_b),
                  mlp_W1, row(mlp_b1), row(mlp_ln_g), row(mlp_ln_b),
                  mw2p, mb2p, interpret=interpret)
    return logits[:, :OUT]
